# baseline (speedup 1.0000x reference)
import functools

import jax
import jax.numpy as jnp
from jax import lax
from jax.experimental import pallas as pl
from jax.experimental.pallas import tpu as pltpu
from jax.experimental.pallas import tpu_sc as plsc

N = 50000
E = 800000
H = 100
HP = 128
R = 200
B = 8

NC = 2
NS = 16
NW = NC * NS

W1 = 40
EPW = E // NW
NWIN1 = EPW // W1


@functools.cache
def _get_sc_gather():
    mesh = plsc.VectorSubcoreMesh(core_axis_name="c", subcore_axis_name="s",
                                  num_cores=NC, num_subcores=NS)

    @functools.partial(
        pl.kernel,
        out_type=jax.ShapeDtypeStruct((E, HP), jnp.float32),
        mesh=mesh,
        scratch_types=[
            pltpu.VMEM((EPW,), jnp.int32),
            pltpu.VMEM((W1, HP), jnp.float32),
            pltpu.VMEM((W1, HP), jnp.float32),
            pltpu.SemaphoreType.DMA,
            pltpu.SemaphoreType.DMA,
            pltpu.SemaphoreType.DMA,
            pltpu.SemaphoreType.DMA,
        ],
    )
    def _sc_gather(src_hbm, h_hbm, z_hbm, sidx_v, rows0, rows1,
                   gsem0, gsem1, wsem0, wsem1):
        wid = lax.axis_index("s") * NC + lax.axis_index("c")
        ebase = wid * EPW
        pltpu.sync_copy(src_hbm.at[pl.ds(ebase, EPW)], sidx_v)

        def gdesc(j, rows, sem):
            return pltpu.make_async_copy(
                h_hbm.at[sidx_v.at[pl.ds(j * W1, W1)]], rows, sem)

        def wdesc(j, rows, sem):
            return pltpu.make_async_copy(
                rows, z_hbm.at[pl.ds(ebase + j * W1, W1), :], sem)

        gdesc(0, rows0, gsem0).start()

        def body(t, carry):
            j1 = 2 * t + 1
            j2 = 2 * t + 2

            @pl.when(t >= 1)
            def _():
                wdesc(j1 - 2, rows1, wsem1).wait()
            gdesc(j1, rows1, gsem1).start()
            gdesc(j1 - 1, rows0, gsem0).wait()
            wdesc(j1 - 1, rows0, wsem0).start()

            wdesc(j1 - 1, rows0, wsem0).wait()
            gdesc(j2, rows0, gsem0).start()
            gdesc(j1, rows1, gsem1).wait()
            wdesc(j1, rows1, wsem1).start()
            return carry

        lax.fori_loop(0, (NWIN1 - 1) // 2, body, 0)
        last = NWIN1 - 1
        gdesc(last, rows0, gsem0).wait()
        wdesc(last, rows0, wsem0).start()
        wdesc(last - 1, rows1, wsem1).wait()
        wdesc(last, rows0, wsem0).wait()

    return _sc_gather


W3 = 80
EPT = E // NS
NWIN3 = EPT // W3
CHUNK = 12512
TRASH = 512
ACC_ROWS = CHUNK + TRASH
ZSTRIP = 816
ZLAST = ACC_ROWS - 15 * ZSTRIP
WSTRIP = 784
WLAST = CHUNK - 15 * WSTRIP
WLAST3 = N - 3 * CHUNK - 15 * WSTRIP


@functools.cache
def _get_sc_scatter():
    mesh = plsc.VectorSubcoreMesh(core_axis_name="c", subcore_axis_name="s",
                                  num_cores=NC, num_subcores=NS)

    @functools.partial(
        pl.kernel,
        out_type=jax.ShapeDtypeStruct((N, HP), jnp.float32),
        mesh=mesh,
        scratch_types=[
            pltpu.VMEM((2, W3), jnp.int32),
            pltpu.VMEM((W3, HP), jnp.float32),
            pltpu.VMEM((W3, HP), jnp.float32),
            pltpu.SemaphoreType.DMA,
            pltpu.SemaphoreType.DMA,
            pltpu.SemaphoreType.DMA,
            pltpu.SemaphoreType.DMA,
            pltpu.VMEM_SHARED((ACC_ROWS, HP), jnp.float32),
        ],
    )
    def _sc_scatter(dst3_hbm, msg_hbm, zrows_hbm, agg_hbm,
                    didxw2, upd0, upd1, msem0, msem1, dsem0, dsem1, acc_sh):
        c = lax.axis_index("c")
        s = lax.axis_index("s")

        def mdesc(j, upd, sem):
            return pltpu.make_async_copy(
                msg_hbm.at[pl.ds(s * EPT + j * W3, W3), :], upd, sem)

        def ddesc(j, buf, sem):
            return pltpu.make_async_copy(
                dst3_hbm.at[s, pl.ds(j, 1), :],
                didxw2.at[pl.ds(buf, 1), :], sem)

        for gi in range(2):
            lo = (2 * c + gi) * CHUNK

            @pl.when(s < NS - 1)
            def _():
                pltpu.sync_copy(zrows_hbm,
                                acc_sh.at[pl.ds(s * ZSTRIP, ZSTRIP), :])

            @pl.when(s == NS - 1)
            def _():
                pltpu.sync_copy(zrows_hbm.at[pl.ds(0, ZLAST)],
                                acc_sh.at[pl.ds(s * ZSTRIP, ZLAST), :])

            plsc.subcore_barrier()

            def compute(buf):
                for k in range(W3 // 16):
                    d = didxw2[buf, pl.ds(k * 16, 16)]
                    inn = (d >= lo) & (d < lo + CHUNK)
                    didxw2[buf, pl.ds(k * 16, 16)] = jnp.where(
                        inn, d - lo, CHUNK + (d & (TRASH - 1)))

            ddesc(0, 0, dsem0).start()
            mdesc(0, upd0, msem0).start()

            def body(t, carry):
                j1 = 2 * t
                j2 = 2 * t + 1
                ddesc(j1 + 1, 1, dsem1).start()
                mdesc(j1 + 1, upd1, msem1).start()
                ddesc(j1, 0, dsem0).wait()
                compute(0)
                mdesc(j1, upd0, msem0).wait()
                pltpu.sync_copy(upd0, acc_sh.at[didxw2.at[0]], add=True)
                ddesc(j2 + 1, 0, dsem0).start()
                mdesc(j2 + 1, upd0, msem0).start()
                ddesc(j2, 1, dsem1).wait()
                compute(1)
                mdesc(j2, upd1, msem1).wait()
                pltpu.sync_copy(upd1, acc_sh.at[didxw2.at[1]], add=True)
                return carry

            lax.fori_loop(0, (NWIN3 - 1) // 2, body, 0)
            ddesc(NWIN3 - 1, 0, dsem0).wait()
            compute(0)
            mdesc(NWIN3 - 1, upd0, msem0).wait()
            pltpu.sync_copy(upd0, acc_sh.at[didxw2.at[0]], add=True)
            plsc.subcore_barrier()

            @pl.when(s < NS - 1)
            def _():
                pltpu.sync_copy(acc_sh.at[pl.ds(s * WSTRIP, WSTRIP), :],
                                agg_hbm.at[pl.ds(lo + s * WSTRIP, WSTRIP), :])

            if gi == 0:
                @pl.when(s == NS - 1)
                def _():
                    pltpu.sync_copy(
                        acc_sh.at[pl.ds(s * WSTRIP, WLAST), :],
                        agg_hbm.at[pl.ds(lo + s * WSTRIP, WLAST), :])
            else:
                @pl.when((s == NS - 1) & (c == 0))
                def _():
                    pltpu.sync_copy(
                        acc_sh.at[pl.ds(s * WSTRIP, WLAST), :],
                        agg_hbm.at[pl.ds(lo + s * WSTRIP, WLAST), :])

                @pl.when((s == NS - 1) & (c == 1))
                def _():
                    pltpu.sync_copy(
                        acc_sh.at[pl.ds(s * WSTRIP, WLAST3), :],
                        agg_hbm.at[pl.ds(lo + s * WSTRIP, WLAST3), :])

            plsc.subcore_barrier()

    return _sc_scatter


WE = 1000
NEB = E // WE


def _tc_edge_body(rel_ref, z_ref, wc_ref, bs_ref, out_ref):
    rel = rel_ref[0, 0, :]
    onehot = (rel[:, None] == lax.broadcasted_iota(jnp.int32, (WE, R), 1))
    coef = jnp.dot(onehot.astype(jnp.bfloat16), wc_ref[...],
                   preferred_element_type=jnp.float32)
    zb = z_ref[:, :H].astype(jnp.bfloat16)
    cb = coef.astype(jnp.bfloat16)
    y = jnp.concatenate([cb[:, b:b + 1] * zb for b in range(B)], axis=1)
    acc = jnp.dot(y, bs_ref[...], preferred_element_type=jnp.float32)
    out_ref[...] = jnp.concatenate(
        [acc, jnp.zeros((WE, HP - H), jnp.float32)], axis=1)


def _tc_edge(rel3, z, w_comp, bases_st):
    return pl.pallas_call(
        _tc_edge_body,
        grid=(NEB,),
        in_specs=[
            pl.BlockSpec((1, 1, WE), lambda i: (i, 0, 0)),
            pl.BlockSpec((WE, HP), lambda i: (i, 0)),
            pl.BlockSpec((R, B), lambda i: (0, 0)),
            pl.BlockSpec((B * H, H), lambda i: (0, 0)),
        ],
        out_specs=pl.BlockSpec((WE, HP), lambda i: (i, 0)),
        out_shape=jax.ShapeDtypeStruct((E, HP), jnp.float32),
    )(rel3, z, w_comp, bases_st)


WN = 1000
NNB = N // WN


def _tc_final_body(agg_ref, h_ref, lw_ref, bias_ref, out_ref):
    hs = jnp.dot(h_ref[:, :H].astype(jnp.bfloat16), lw_ref[...],
                 preferred_element_type=jnp.float32)
    out_ref[...] = jnp.maximum(hs + agg_ref[...] + bias_ref[...], 0.0)


def _tc_final(agg, h, lw_pad, bias_pad):
    return pl.pallas_call(
        _tc_final_body,
        grid=(NNB,),
        in_specs=[
            pl.BlockSpec((WN, HP), lambda i: (i, 0)),
            pl.BlockSpec((WN, HP), lambda i: (i, 0)),
            pl.BlockSpec((H, HP), lambda i: (0, 0)),
            pl.BlockSpec((1, HP), lambda i: (0, 0)),
        ],
        out_specs=pl.BlockSpec((WN, HP), lambda i: (i, 0)),
        out_shape=jax.ShapeDtypeStruct((N, HP), jnp.float32),
    )(agg, h, lw_pad, bias_pad)


def _layer(h, src, dst3, rel3, zrows, bases_st, w_comp, lw_pad, bias_pad):
    z = _get_sc_gather()(src, h)
    msg = _tc_edge(rel3, z, w_comp, bases_st)
    agg = _get_sc_scatter()(dst3, msg, zrows)
    return _tc_final(agg, h, lw_pad, bias_pad)


def kernel(edge_index, rel_type, entity_embedding,
           bases_0, w_comp_0, loop_w_0, bias_0,
           bases_1, w_comp_1, loop_w_1, bias_1):
    src = edge_index[0]
    dst3 = edge_index[1].reshape(NS, NWIN3, W3)
    rel3 = rel_type.reshape(NEB, 1, WE)
    zrows = jnp.zeros((ZSTRIP, HP), jnp.float32)
    h = jnp.pad(entity_embedding, ((0, 0), (0, HP - H)))
    for bases, w_comp, loop_w, bias in (
            (bases_0, w_comp_0, loop_w_0, bias_0),
            (bases_1, w_comp_1, loop_w_1, bias_1)):
        bases_st = bases.reshape(B * H, H).astype(jnp.bfloat16)
        lw_pad = jnp.pad(loop_w, ((0, 0), (0, HP - H))).astype(jnp.bfloat16)
        bias_pad = jnp.pad(bias, (0, HP - H)).reshape(1, HP)
        h = _layer(h, src, dst3, rel3, zrows, bases_st,
                   w_comp.astype(jnp.bfloat16), lw_pad, bias_pad)
    return h[:, :H]


# speedup vs baseline: 1.4407x; 1.4407x over previous
import functools

import jax
import jax.numpy as jnp
from jax import lax
from jax.experimental import pallas as pl
from jax.experimental.pallas import tpu as pltpu
from jax.experimental.pallas import tpu_sc as plsc

N = 50000
E = 800000
H = 100
HP = 128
R = 200
B = 8

NC = 2
NS = 16
NW = NC * NS

EA = 416000
EB = E - EA


@functools.cache
def _get_sc_gather(ne, w1):
    epw = ne // NW
    nwin = epw // w1
    assert nwin % 2 == 1
    mesh = plsc.VectorSubcoreMesh(core_axis_name="c", subcore_axis_name="s",
                                  num_cores=NC, num_subcores=NS)
    W1 = w1

    @functools.partial(
        pl.kernel,
        out_type=jax.ShapeDtypeStruct((ne, HP), jnp.float32),
        mesh=mesh,
        scratch_types=[
            pltpu.VMEM((epw,), jnp.int32),
            pltpu.VMEM((W1, HP), jnp.float32),
            pltpu.VMEM((W1, HP), jnp.float32),
            pltpu.SemaphoreType.DMA,
            pltpu.SemaphoreType.DMA,
            pltpu.SemaphoreType.DMA,
            pltpu.SemaphoreType.DMA,
        ],
    )
    def _sc_gather(src_hbm, h_hbm, z_hbm, sidx_v, rows0, rows1,
                   gsem0, gsem1, wsem0, wsem1):
        wid = lax.axis_index("s") * NC + lax.axis_index("c")
        ebase = wid * epw
        pltpu.sync_copy(src_hbm.at[pl.ds(ebase, epw)], sidx_v)

        def gdesc(j, rows, sem):
            return pltpu.make_async_copy(
                h_hbm.at[sidx_v.at[pl.ds(j * W1, W1)]], rows, sem)

        def wdesc(j, rows, sem):
            return pltpu.make_async_copy(
                rows, z_hbm.at[pl.ds(ebase + j * W1, W1), :], sem)

        gdesc(0, rows0, gsem0).start()

        def body(t, carry):
            j1 = 2 * t + 1
            j2 = 2 * t + 2

            @pl.when(t >= 1)
            def _():
                wdesc(j1 - 2, rows1, wsem1).wait()
            gdesc(j1, rows1, gsem1).start()
            gdesc(j1 - 1, rows0, gsem0).wait()
            wdesc(j1 - 1, rows0, wsem0).start()

            wdesc(j1 - 1, rows0, wsem0).wait()
            gdesc(j2, rows0, gsem0).start()
            gdesc(j1, rows1, gsem1).wait()
            wdesc(j1, rows1, wsem1).start()
            return carry

        lax.fori_loop(0, (nwin - 1) // 2, body, 0)
        last = nwin - 1
        gdesc(last, rows0, gsem0).wait()
        wdesc(last, rows0, wsem0).start()
        wdesc(last - 1, rows1, wsem1).wait()
        wdesc(last, rows0, wsem0).wait()

    return _sc_gather


CHUNK = 12512
TRASH = 512
ACC_ROWS = CHUNK + TRASH
ZSTRIP = 816
ZLAST = ACC_ROWS - 15 * ZSTRIP
WSTRIP = 784
WLAST = CHUNK - 15 * WSTRIP
WLAST3 = N - 3 * CHUNK - 15 * WSTRIP


@functools.cache
def _get_sc_scatter(ne, w3, init_from_agg):
    ept = ne // NS
    nwin = ept // w3
    assert nwin % 2 == 1 and w3 % 16 == 0
    mesh = plsc.VectorSubcoreMesh(core_axis_name="c", subcore_axis_name="s",
                                  num_cores=NC, num_subcores=NS)
    W3 = w3

    @functools.partial(
        pl.kernel,
        out_type=jax.ShapeDtypeStruct((N, HP), jnp.float32),
        mesh=mesh,
        scratch_types=[
            pltpu.VMEM((2, W3), jnp.int32),
            pltpu.VMEM((W3, HP), jnp.float32),
            pltpu.VMEM((W3, HP), jnp.float32),
            pltpu.SemaphoreType.DMA,
            pltpu.SemaphoreType.DMA,
            pltpu.SemaphoreType.DMA,
            pltpu.SemaphoreType.DMA,
            pltpu.VMEM_SHARED((ACC_ROWS, HP), jnp.float32),
        ],
    )
    def _sc_scatter(dst3_hbm, msg_hbm, init_hbm, agg_hbm,
                    didxw2, upd0, upd1, msem0, msem1, dsem0, dsem1, acc_sh):
        c = lax.axis_index("c")
        s = lax.axis_index("s")

        def mdesc(j, upd, sem):
            return pltpu.make_async_copy(
                msg_hbm.at[pl.ds(s * ept + j * W3, W3), :], upd, sem)

        def ddesc(j, buf, sem):
            return pltpu.make_async_copy(
                dst3_hbm.at[s, pl.ds(j, 1), :],
                didxw2.at[pl.ds(buf, 1), :], sem)

        for gi in range(2):
            lo = (2 * c + gi) * CHUNK

            if not init_from_agg:
                @pl.when(s < NS - 1)
                def _():
                    pltpu.sync_copy(init_hbm,
                                    acc_sh.at[pl.ds(s * ZSTRIP, ZSTRIP), :])

                @pl.when(s == NS - 1)
                def _():
                    pltpu.sync_copy(init_hbm.at[pl.ds(0, ZLAST)],
                                    acc_sh.at[pl.ds(s * ZSTRIP, ZLAST), :])
            else:
                @pl.when(s < NS - 1)
                def _():
                    pltpu.sync_copy(
                        init_hbm.at[pl.ds(lo + s * WSTRIP, WSTRIP), :],
                        acc_sh.at[pl.ds(s * WSTRIP, WSTRIP), :])

                if gi == 0:
                    @pl.when(s == NS - 1)
                    def _():
                        pltpu.sync_copy(
                            init_hbm.at[pl.ds(lo + s * WSTRIP, WLAST), :],
                            acc_sh.at[pl.ds(s * WSTRIP, WLAST), :])
                else:
                    @pl.when((s == NS - 1) & (c == 0))
                    def _():
                        pltpu.sync_copy(
                            init_hbm.at[pl.ds(lo + s * WSTRIP, WLAST), :],
                            acc_sh.at[pl.ds(s * WSTRIP, WLAST), :])

                    @pl.when((s == NS - 1) & (c == 1))
                    def _():
                        pltpu.sync_copy(
                            init_hbm.at[pl.ds(lo + s * WSTRIP, WLAST3), :],
                            acc_sh.at[pl.ds(s * WSTRIP, WLAST3), :])

            plsc.subcore_barrier()

            def compute(buf):
                for k in range(W3 // 16):
                    d = didxw2[buf, pl.ds(k * 16, 16)]
                    inn = (d >= lo) & (d < lo + CHUNK)
                    didxw2[buf, pl.ds(k * 16, 16)] = jnp.where(
                        inn, d - lo, CHUNK + (d & (TRASH - 1)))

            ddesc(0, 0, dsem0).start()
            mdesc(0, upd0, msem0).start()

            def body(t, carry):
                j1 = 2 * t
                j2 = 2 * t + 1
                ddesc(j1 + 1, 1, dsem1).start()
                mdesc(j1 + 1, upd1, msem1).start()
                ddesc(j1, 0, dsem0).wait()
                compute(0)
                mdesc(j1, upd0, msem0).wait()
                pltpu.sync_copy(upd0, acc_sh.at[didxw2.at[0]], add=True)
                ddesc(j2 + 1, 0, dsem0).start()
                mdesc(j2 + 1, upd0, msem0).start()
                ddesc(j2, 1, dsem1).wait()
                compute(1)
                mdesc(j2, upd1, msem1).wait()
                pltpu.sync_copy(upd1, acc_sh.at[didxw2.at[1]], add=True)
                return carry

            lax.fori_loop(0, (nwin - 1) // 2, body, 0)
            ddesc(nwin - 1, 0, dsem0).wait()
            compute(0)
            mdesc(nwin - 1, upd0, msem0).wait()
            pltpu.sync_copy(upd0, acc_sh.at[didxw2.at[0]], add=True)
            plsc.subcore_barrier()

            @pl.when(s < NS - 1)
            def _():
                pltpu.sync_copy(acc_sh.at[pl.ds(s * WSTRIP, WSTRIP), :],
                                agg_hbm.at[pl.ds(lo + s * WSTRIP, WSTRIP), :])

            if gi == 0:
                @pl.when(s == NS - 1)
                def _():
                    pltpu.sync_copy(
                        acc_sh.at[pl.ds(s * WSTRIP, WLAST), :],
                        agg_hbm.at[pl.ds(lo + s * WSTRIP, WLAST), :])
            else:
                @pl.when((s == NS - 1) & (c == 0))
                def _():
                    pltpu.sync_copy(
                        acc_sh.at[pl.ds(s * WSTRIP, WLAST), :],
                        agg_hbm.at[pl.ds(lo + s * WSTRIP, WLAST), :])

                @pl.when((s == NS - 1) & (c == 1))
                def _():
                    pltpu.sync_copy(
                        acc_sh.at[pl.ds(s * WSTRIP, WLAST3), :],
                        agg_hbm.at[pl.ds(lo + s * WSTRIP, WLAST3), :])

            plsc.subcore_barrier()

    return _sc_scatter


WE = 640
NEB = E // WE


def _tc_edge_body(rel_ref, z_ref, wc_ref, bs_ref, out_ref):
    rel = rel_ref[0, 0, :]
    onehot = (rel[:, None] == lax.broadcasted_iota(jnp.int32, (WE, R), 1))
    coef = jnp.dot(onehot.astype(jnp.bfloat16), wc_ref[...],
                   preferred_element_type=jnp.float32)
    zb = z_ref[:, :H].astype(jnp.bfloat16)
    cb = coef.astype(jnp.bfloat16)
    acc = jnp.zeros((WE, H), jnp.float32)
    for b in range(B):
        acc = acc + jnp.dot(cb[:, b:b + 1] * zb,
                            bs_ref[pl.ds(b * H, H), :],
                            preferred_element_type=jnp.float32)
    out_ref[...] = jnp.concatenate(
        [acc, jnp.zeros((WE, HP - H), jnp.float32)], axis=1)


def _tc_edge(rel3, z, w_comp, bases_st):
    neb = z.shape[0] // WE
    return pl.pallas_call(
        _tc_edge_body,
        grid=(neb,),
        in_specs=[
            pl.BlockSpec((1, 1, WE), lambda i: (i, 0, 0)),
            pl.BlockSpec((WE, HP), lambda i: (i, 0)),
            pl.BlockSpec((R, B), lambda i: (0, 0)),
            pl.BlockSpec((B * H, H), lambda i: (0, 0)),
        ],
        out_specs=pl.BlockSpec((WE, HP), lambda i: (i, 0)),
        out_shape=jax.ShapeDtypeStruct((z.shape[0], HP), jnp.float32),
    )(rel3, z, w_comp, bases_st)


WN = 1000
NNB = N // WN


def _tc_final_body(agg_ref, h_ref, lw_ref, bias_ref, out_ref):
    hs = jnp.dot(h_ref[:, :H].astype(jnp.bfloat16), lw_ref[...],
                 preferred_element_type=jnp.float32)
    out_ref[...] = jnp.maximum(hs + agg_ref[...] + bias_ref[...], 0.0)


def _tc_final(agg, h, lw_pad, bias_pad):
    return pl.pallas_call(
        _tc_final_body,
        grid=(NNB,),
        in_specs=[
            pl.BlockSpec((WN, HP), lambda i: (i, 0)),
            pl.BlockSpec((WN, HP), lambda i: (i, 0)),
            pl.BlockSpec((H, HP), lambda i: (0, 0)),
            pl.BlockSpec((1, HP), lambda i: (0, 0)),
        ],
        out_specs=pl.BlockSpec((WN, HP), lambda i: (i, 0)),
        out_shape=jax.ShapeDtypeStruct((N, HP), jnp.float32),
    )(agg, h, lw_pad, bias_pad)


W1A, W1B = 40, 32
W3A, W3B = 80, 64


def _layer(h, halves, zrows, bases_st, w_comp, lw_pad, bias_pad):
    (srcA, dst3A, rel3A), (srcB, dst3B, rel3B) = halves
    zA = _get_sc_gather(EA, W1A)(srcA, h)
    msgA = _tc_edge(rel3A, zA, w_comp, bases_st)
    zB = _get_sc_gather(EB, W1B)(srcB, h)
    msgB = _tc_edge(rel3B, zB, w_comp, bases_st)
    aggP = _get_sc_scatter(EA, W3A, False)(dst3A, msgA, zrows)
    agg = _get_sc_scatter(EB, W3B, True)(dst3B, msgB, aggP)
    return _tc_final(agg, h, lw_pad, bias_pad)


def kernel(edge_index, rel_type, entity_embedding,
           bases_0, w_comp_0, loop_w_0, bias_0,
           bases_1, w_comp_1, loop_w_1, bias_1):
    src = edge_index[0]
    dst = edge_index[1]
    halves = (
        (src[:EA], dst[:EA].reshape(NS, EA // NS // W3A, W3A),
         rel_type[:EA].reshape(EA // WE, 1, WE)),
        (src[EA:], dst[EA:].reshape(NS, EB // NS // W3B, W3B),
         rel_type[EA:].reshape(EB // WE, 1, WE)),
    )
    zrows = jnp.zeros((ZSTRIP, HP), jnp.float32)
    h = jnp.pad(entity_embedding, ((0, 0), (0, HP - H)))
    for bases, w_comp, loop_w, bias in (
            (bases_0, w_comp_0, loop_w_0, bias_0),
            (bases_1, w_comp_1, loop_w_1, bias_1)):
        bases_st = bases.reshape(B * H, H).astype(jnp.bfloat16)
        lw_pad = jnp.pad(loop_w, ((0, 0), (0, HP - H))).astype(jnp.bfloat16)
        bias_pad = jnp.pad(bias, (0, HP - H)).reshape(1, HP)
        h = _layer(h, halves, zrows, bases_st,
                   w_comp.astype(jnp.bfloat16), lw_pad, bias_pad)
    return h[:, :H]
